# baseline (speedup 1.0000x reference)
import functools

import jax
import jax.numpy as jnp
from jax import lax
from jax.experimental import pallas as pl
from jax.experimental.pallas import tpu as pltpu
from jax.experimental.pallas import tpu_sc as plsc

N = 10000
E = 160000
F_IN = 128
D = 32
B = 64

N_PAD = 10240
E_PAD = 163840
NTILES = 32
EPT = E_PAD // NTILES
CHUNK = 128
NCHUNK = EPT // CHUNK
NB = 1280
NGB = N_PAD // NB
EGB = E_PAD // NB
ROWS_PER_SUB = N_PAD // 16

def _sc_mesh():
  return plsc.VectorSubcoreMesh(
      core_axis_name="c", subcore_axis_name="s", num_cores=2, num_subcores=16)


def _gather_body(h_hbm, src_hbm, xj_hbm, idx_v, row_v, sem):
  cid = lax.axis_index("c")
  sid = lax.axis_index("s")
  wid = cid * 16 + sid
  pltpu.sync_copy(src_hbm.at[wid], idx_v)

  def chunk(j, carry):
    pltpu.async_copy(h_hbm.at[idx_v.at[j]], row_v, sem).wait()
    pltpu.sync_copy(row_v, xj_hbm.at[pl.ds(wid * EPT + j * CHUNK, CHUNK)])
    return carry

  lax.fori_loop(0, NCHUNK, chunk, 0)


@functools.lru_cache(maxsize=None)
def _sc_gather_kernel():
  return pl.kernel(
      _gather_body,
      out_type=jax.ShapeDtypeStruct((E_PAD, D), jnp.float32),
      mesh=_sc_mesh(),
      compiler_params=pltpu.CompilerParams(use_tc_tiling_on_sc=False),
      scratch_types=[
          pltpu.VMEM((NCHUNK, CHUNK), jnp.int32),
          pltpu.VMEM((CHUNK, D), jnp.float32),
          pltpu.SemaphoreType.DMA,
      ],
  )


def _sc_gather(h, src_t):
  return _sc_gather_kernel()(h, src_t)


def _scatter_body(msg_hbm, dst_hbm, zagg_hbm, zdeg_hbm, ones_hbm,
                  aggp_hbm, degp_hbm, idx_v, msg_v, ones_v, agg_sh, deg_sh,
                  with_deg):
  cid = lax.axis_index("c")
  sid = lax.axis_index("s")
  wid = cid * 16 + sid
  pltpu.sync_copy(zagg_hbm, agg_sh.at[pl.ds(sid * ROWS_PER_SUB, ROWS_PER_SUB)])
  if with_deg:
    pltpu.sync_copy(zdeg_hbm,
                    deg_sh.at[pl.ds(sid * ROWS_PER_SUB, ROWS_PER_SUB)])
    pltpu.sync_copy(ones_hbm, ones_v)
  pltpu.sync_copy(dst_hbm.at[wid], idx_v)
  plsc.subcore_barrier()

  def chunk(j, carry):
    pltpu.sync_copy(msg_hbm.at[pl.ds(wid * EPT + j * CHUNK, CHUNK)], msg_v)
    pltpu.sync_copy(msg_v, agg_sh.at[idx_v.at[j]], add=True)
    if with_deg:
      pltpu.sync_copy(ones_v, deg_sh.at[idx_v.at[j]], add=True)
    return carry

  lax.fori_loop(0, NCHUNK, chunk, 0)
  plsc.subcore_barrier()
  sl = pl.ds(sid * ROWS_PER_SUB, ROWS_PER_SUB)
  pltpu.sync_copy(agg_sh.at[sl], aggp_hbm.at[cid].at[sl])
  if with_deg:
    pltpu.sync_copy(deg_sh.at[sl], degp_hbm.at[cid].at[sl])


@functools.lru_cache(maxsize=None)
def _make_scatter(with_deg):
  return pl.kernel(
      functools.partial(_scatter_body, with_deg=with_deg),
      out_type=(
          jax.ShapeDtypeStruct((2, N_PAD, D), jnp.float32),
          jax.ShapeDtypeStruct((2, N_PAD, 16), jnp.float32),
      ),
      mesh=_sc_mesh(),
      compiler_params=pltpu.CompilerParams(use_tc_tiling_on_sc=False),
      scratch_types=[
          pltpu.VMEM((NCHUNK, CHUNK), jnp.int32),
          pltpu.VMEM((CHUNK, D), jnp.float32),
          pltpu.VMEM((CHUNK, 16), jnp.float32),
          pltpu.VMEM_SHARED((N_PAD, D), jnp.float32),
          pltpu.VMEM_SHARED((N_PAD, 16), jnp.float32),
      ],
  )


def _sc_scatter_deg(msg, dst_t, zagg, zdeg, ones):
  return _make_scatter(True)(msg, dst_t, zagg, zdeg, ones)


def _sc_scatter(msg, dst_t, zagg, zdeg, ones):
  return _make_scatter(False)(msg, dst_t, zagg, zdeg, ones)


def _lin0_body(x_ref, w0t_ref, b0_ref, out_ref):
  out_ref[...] = jax.nn.relu(
      jnp.dot(x_ref[...], w0t_ref[...], preferred_element_type=jnp.float32)
      + b0_ref[...])


def _msg_body(ea_ref, xj_ref, w1t_ref, b1_ref, w2t_ref, b2_ref, r_ref, s_ref,
              msg_ref):
  he = jax.nn.relu(
      jnp.dot(ea_ref[...], w1t_ref[...], preferred_element_type=jnp.float32)
      + b1_ref[...])
  we = jnp.dot(he.astype(jnp.bfloat16), w2t_ref[...],
               preferred_element_type=jnp.float32) + b2_ref[...]
  xj_b = jnp.dot(xj_ref[...], r_ref[...], preferred_element_type=jnp.float32)
  msg_ref[...] = jnp.dot(xj_b * we, s_ref[...],
                         preferred_element_type=jnp.float32)


def _gru_body(aggp_ref, degp_ref, h_ref, bconv_ref, wih_ref, whh_ref,
              bih_ref, bhh_ref, out_ref):
  agg = aggp_ref[0] + aggp_ref[1]
  deg = degp_ref[0][:, 0:1] + degp_ref[1][:, 0:1]
  deg = jnp.maximum(deg, 1.0)
  m = jax.nn.relu(agg / deg + bconv_ref[...])
  h = h_ref[...]
  gi = jnp.dot(m, wih_ref[...], preferred_element_type=jnp.float32) + bih_ref[...]
  gh = jnp.dot(h, whh_ref[...], preferred_element_type=jnp.float32) + bhh_ref[...]
  r = jax.nn.sigmoid(gi[:, 0:D] + gh[:, 0:D])
  z = jax.nn.sigmoid(gi[:, D:2 * D] + gh[:, D:2 * D])
  n = jnp.tanh(gi[:, 2 * D:3 * D] + r * gh[:, 2 * D:3 * D])
  out_ref[...] = (1.0 - z) * n + z * h


def _set2set_body(h_ref, batch_ref, wih_ref, whh_ref, bih_ref, bhh_ref,
                  q_ref):
  out = h_ref[...]
  onehot = (batch_ref[...] == lax.broadcasted_iota(
      jnp.int32, (N_PAD, B), 1)).astype(jnp.float32)
  valid = jnp.sum(onehot, axis=1, keepdims=True)
  hl = jnp.zeros((B, D), jnp.float32)
  cl = jnp.zeros((B, D), jnp.float32)
  q_star = jnp.zeros((B, 2 * D), jnp.float32)
  for _ in range(3):
    gates = (jnp.dot(q_star, wih_ref[...], preferred_element_type=jnp.float32)
             + bih_ref[...]
             + jnp.dot(hl, whh_ref[...], preferred_element_type=jnp.float32)
             + bhh_ref[...])
    gi_ = gates[:, 0:D]
    gf_ = gates[:, D:2 * D]
    gg_ = gates[:, 2 * D:3 * D]
    go_ = gates[:, 3 * D:4 * D]
    cl = jax.nn.sigmoid(gf_) * cl + jax.nn.sigmoid(gi_) * jnp.tanh(gg_)
    hl = jax.nn.sigmoid(go_) * jnp.tanh(cl)
    qb = jnp.dot(onehot, hl, preferred_element_type=jnp.float32)
    e = jnp.sum(out * qb, axis=1, keepdims=True)
    x_masked = jnp.where(onehot > 0.0, e, -1e30)
    emax = jnp.max(x_masked, axis=0, keepdims=True)
    emax_b = jnp.sum(onehot * emax, axis=1, keepdims=True)
    a = jnp.exp(e - emax_b) * valid
    s = jnp.sum(onehot * a, axis=0, keepdims=True)
    s_b = jnp.sum(onehot * s, axis=1, keepdims=True)
    a = a / (s_b + 1e-16)
    rvec = lax.dot_general(onehot, a * out, (((0,), (0,)), ((), ())),
                           preferred_element_type=jnp.float32)
    q_star = jnp.concatenate([hl, rvec], axis=1)
  q_ref[...] = q_star


def _tc_lin0(x_p, w0t, b0):
  return pl.pallas_call(
      _lin0_body,
      grid=(NGB,),
      in_specs=[
          pl.BlockSpec((NB, F_IN), lambda i: (i, 0)),
          pl.BlockSpec((F_IN, D), lambda i: (0, 0)),
          pl.BlockSpec((1, D), lambda i: (0, 0)),
      ],
      out_specs=pl.BlockSpec((NB, D), lambda i: (i, 0)),
      out_shape=jax.ShapeDtypeStruct((N_PAD, D), jnp.float32),
  )(x_p, w0t, b0)


def _tc_msg(ea_p, xj, w1t, b1, w2t, b2, r_mat, s_mat):
  return pl.pallas_call(
      _msg_body,
      grid=(EGB,),
      in_specs=[
          pl.BlockSpec((NB, 8), lambda i: (i, 0)),
          pl.BlockSpec((NB, D), lambda i: (i, 0)),
          pl.BlockSpec((8, F_IN), lambda i: (0, 0)),
          pl.BlockSpec((1, F_IN), lambda i: (0, 0)),
          pl.BlockSpec((F_IN, D * D), lambda i: (0, 0)),
          pl.BlockSpec((1, D * D), lambda i: (0, 0)),
          pl.BlockSpec((D, D * D), lambda i: (0, 0)),
          pl.BlockSpec((D * D, D), lambda i: (0, 0)),
      ],
      out_specs=pl.BlockSpec((NB, D), lambda i: (i, 0)),
      out_shape=jax.ShapeDtypeStruct((E_PAD, D), jnp.float32),
  )(ea_p, xj, w1t, b1, w2t, b2, r_mat, s_mat)


def _tc_gru(aggp, degp, h, bconv, wih_t, whh_t, bih, bhh):
  return pl.pallas_call(
      _gru_body,
      grid=(NGB,),
      in_specs=[
          pl.BlockSpec((2, NB, D), lambda i: (0, i, 0)),
          pl.BlockSpec((2, NB, 16), lambda i: (0, i, 0)),
          pl.BlockSpec((NB, D), lambda i: (i, 0)),
          pl.BlockSpec((1, D), lambda i: (0, 0)),
          pl.BlockSpec((D, 3 * D), lambda i: (0, 0)),
          pl.BlockSpec((D, 3 * D), lambda i: (0, 0)),
          pl.BlockSpec((1, 3 * D), lambda i: (0, 0)),
          pl.BlockSpec((1, 3 * D), lambda i: (0, 0)),
      ],
      out_specs=pl.BlockSpec((NB, D), lambda i: (i, 0)),
      out_shape=jax.ShapeDtypeStruct((N_PAD, D), jnp.float32),
  )(aggp, degp, h, bconv, wih_t, whh_t, bih, bhh)


def _tc_set2set(h, batch2d, wih_t, whh_t, bih, bhh):
  return pl.pallas_call(
      _set2set_body,
      out_shape=jax.ShapeDtypeStruct((B, 2 * D), jnp.float32),
  )(h, batch2d, wih_t, whh_t, bih, bhh)


def kernel(x, edge_index, edge_attr, batch, W0, b0, W1, b1, W2, b2, b_conv,
           gru_Wih, gru_Whh, gru_bih, gru_bhh, ls_Wih, ls_Whh, ls_bih,
           ls_bhh):
  x_p = jnp.pad(x, ((0, N_PAD - N), (0, 0)))
  batch2d = jnp.pad(batch, (0, N_PAD - N), constant_values=B).reshape(N_PAD, 1)
  ea_p = jnp.pad(edge_attr, ((0, E_PAD - E), (0, 3)))
  src_t = jnp.pad(edge_index[0], (0, E_PAD - E)).reshape(NTILES, NCHUNK, CHUNK)
  dst_t = jnp.pad(edge_index[1], (0, E_PAD - E),
                  constant_values=N).reshape(NTILES, NCHUNK, CHUNK)

  w0t = W0.T
  b0r = b0.reshape(1, D)
  w1t = jnp.pad(W1, ((0, 0), (0, 3))).T
  b1r = b1.reshape(1, F_IN)
  w2t = W2.T.astype(jnp.bfloat16)
  b2r = b2.reshape(1, D * D)
  bconv = b_conv.reshape(1, D)
  gwih_t = gru_Wih.T
  gwhh_t = gru_Whh.T
  gbih = gru_bih.reshape(1, 3 * D)
  gbhh = gru_bhh.reshape(1, 3 * D)
  lwih_t = ls_Wih.T
  lwhh_t = ls_Whh.T
  lbih = ls_bih.reshape(1, 4 * D)
  lbhh = ls_bhh.reshape(1, 4 * D)

  eye = jnp.eye(D, dtype=jnp.float32)
  r_mat = jnp.repeat(eye, D, axis=1)
  s_mat = jnp.tile(eye, (D, 1))

  zagg = jnp.zeros((ROWS_PER_SUB, D), jnp.float32)
  zdeg = jnp.zeros((ROWS_PER_SUB, 16), jnp.float32)
  ones = jnp.ones((CHUNK, 16), jnp.float32)

  h = _tc_lin0(x_p, w0t, b0r)

  degp = None
  for r in range(3):
    xj = _sc_gather(h, src_t)
    msg = _tc_msg(ea_p, xj, w1t, b1r, w2t, b2r, r_mat, s_mat)
    if r == 0:
      aggp, degp = _sc_scatter_deg(msg, dst_t, zagg, zdeg, ones)
    else:
      aggp, _ = _sc_scatter(msg, dst_t, zagg, zdeg, ones)
    h = _tc_gru(aggp, degp, h, bconv, gwih_t, gwhh_t, gbih, gbhh)

  q_star = _tc_set2set(h, batch2d, lwih_t, lwhh_t, lbih, lbhh)
  return q_star, h[:N]


# speedup vs baseline: 1.0752x; 1.0752x over previous
import functools

import jax
import jax.numpy as jnp
from jax import lax
from jax.experimental import pallas as pl
from jax.experimental.pallas import tpu as pltpu
from jax.experimental.pallas import tpu_sc as plsc

N = 10000
E = 160000
F_IN = 128
D = 32
B = 64

N_PAD = 10240
E_PAD = 163840
NTILES = 32
EPT = E_PAD // NTILES
CHUNK = 128
NCHUNK = EPT // CHUNK
NB = 1280
NGB = N_PAD // NB
EGB = E_PAD // NB
ROWS_PER_SUB = N_PAD // 16

def _sc_mesh():
  return plsc.VectorSubcoreMesh(
      core_axis_name="c", subcore_axis_name="s", num_cores=2, num_subcores=16)


NBUF = 4


def _gather_body(h_hbm, src_hbm, xj_hbm, idx_v, row_v, gsem, wsem):
  cid = lax.axis_index("c")
  sid = lax.axis_index("s")
  wid = cid * 16 + sid
  pltpu.sync_copy(src_hbm.at[wid], idx_v)

  def superbatch(k, carry):
    for g in range(2):
      base = (k * 2 + g) * NBUF

      @pl.when(k > 0)
      def _drain():
        for _ in range(NBUF):
          pltpu.make_async_copy(row_v.at[g, 0], xj_hbm.at[pl.ds(0, CHUNK)],
                                wsem).wait()

      descs = [
          pltpu.async_copy(h_hbm.at[idx_v.at[base + b]], row_v.at[g, b], gsem)
          for b in range(NBUF)
      ]
      for dsc in descs:
        dsc.wait()
      for b in range(NBUF):
        pltpu.async_copy(
            row_v.at[g, b],
            xj_hbm.at[pl.ds((wid * NCHUNK + base + b) * CHUNK, CHUNK)], wsem)
    return carry

  lax.fori_loop(0, NCHUNK // (2 * NBUF), superbatch, 0)
  for _ in range(2 * NBUF):
    pltpu.make_async_copy(row_v.at[0, 0], xj_hbm.at[pl.ds(0, CHUNK)],
                          wsem).wait()


@functools.lru_cache(maxsize=None)
def _sc_gather_kernel():
  return pl.kernel(
      _gather_body,
      out_type=jax.ShapeDtypeStruct((E_PAD, D), jnp.float32),
      mesh=_sc_mesh(),
      compiler_params=pltpu.CompilerParams(use_tc_tiling_on_sc=False),
      scratch_types=[
          pltpu.VMEM((NCHUNK, CHUNK), jnp.int32),
          pltpu.VMEM((2, NBUF, CHUNK, D), jnp.float32),
          pltpu.SemaphoreType.DMA,
          pltpu.SemaphoreType.DMA,
      ],
  )


def _sc_gather(h, src_t):
  return _sc_gather_kernel()(h, src_t)


def _scatter_body(msg_hbm, dst_hbm, zagg_hbm, zdeg_hbm, ones_hbm,
                  aggp_hbm, degp_hbm, idx_v, msg_v, ones_v, agg_sh, deg_sh,
                  rsem, with_deg):
  cid = lax.axis_index("c")
  sid = lax.axis_index("s")
  wid = cid * 16 + sid
  pltpu.sync_copy(zagg_hbm, agg_sh.at[pl.ds(sid * ROWS_PER_SUB, ROWS_PER_SUB)])
  if with_deg:
    pltpu.sync_copy(zdeg_hbm,
                    deg_sh.at[pl.ds(sid * ROWS_PER_SUB, ROWS_PER_SUB)])
    pltpu.sync_copy(ones_hbm, ones_v)
  pltpu.sync_copy(dst_hbm.at[wid], idx_v)
  plsc.subcore_barrier()

  for p in range(4):
    pltpu.async_copy(msg_hbm.at[pl.ds((wid * NCHUNK + p) * CHUNK, CHUNK)],
                     msg_v.at[p], rsem)

  def quad(k, carry):
    for p in range(4):
      j = 4 * k + p
      pltpu.make_async_copy(msg_hbm.at[pl.ds(0, CHUNK)], msg_v.at[p],
                            rsem).wait()
      pltpu.sync_copy(msg_v.at[p], agg_sh.at[idx_v.at[j]], add=True)
      if with_deg:
        pltpu.sync_copy(ones_v, deg_sh.at[idx_v.at[j]], add=True)

      @pl.when(j + 4 < NCHUNK)
      def _prefetch():
        pltpu.async_copy(
            msg_hbm.at[pl.ds((wid * NCHUNK + j + 4) * CHUNK, CHUNK)],
            msg_v.at[p], rsem)
    return carry

  lax.fori_loop(0, NCHUNK // 4, quad, 0)
  plsc.subcore_barrier()
  sl = pl.ds(sid * ROWS_PER_SUB, ROWS_PER_SUB)
  pltpu.sync_copy(agg_sh.at[sl], aggp_hbm.at[cid].at[sl])
  if with_deg:
    pltpu.sync_copy(deg_sh.at[sl], degp_hbm.at[cid].at[sl])


@functools.lru_cache(maxsize=None)
def _make_scatter(with_deg):
  return pl.kernel(
      functools.partial(_scatter_body, with_deg=with_deg),
      out_type=(
          jax.ShapeDtypeStruct((2, N_PAD, D), jnp.float32),
          jax.ShapeDtypeStruct((2, N_PAD, 16), jnp.float32),
      ),
      mesh=_sc_mesh(),
      compiler_params=pltpu.CompilerParams(use_tc_tiling_on_sc=False),
      scratch_types=[
          pltpu.VMEM((NCHUNK, CHUNK), jnp.int32),
          pltpu.VMEM((4, CHUNK, D), jnp.float32),
          pltpu.VMEM((CHUNK, 16), jnp.float32),
          pltpu.VMEM_SHARED((N_PAD, D), jnp.float32),
          pltpu.VMEM_SHARED((N_PAD, 16), jnp.float32),
          pltpu.SemaphoreType.DMA,
      ],
  )


def _sc_scatter_deg(msg, dst_t, zagg, zdeg, ones):
  return _make_scatter(True)(msg, dst_t, zagg, zdeg, ones)


def _sc_scatter(msg, dst_t, zagg, zdeg, ones):
  return _make_scatter(False)(msg, dst_t, zagg, zdeg, ones)


def _lin0_body(x_ref, w0t_ref, b0_ref, out_ref):
  out_ref[...] = jax.nn.relu(
      jnp.dot(x_ref[...], w0t_ref[...], preferred_element_type=jnp.float32)
      + b0_ref[...])


def _msg_body(ea_ref, xj_ref, w1t_ref, b1_ref, w2t_ref, b2_ref, r_ref, s_ref,
              msg_ref):
  he = jax.nn.relu(
      jnp.dot(ea_ref[...], w1t_ref[...], preferred_element_type=jnp.float32)
      + b1_ref[...])
  we = jnp.dot(he.astype(jnp.bfloat16), w2t_ref[...],
               preferred_element_type=jnp.float32) + b2_ref[...]
  xj_b = jnp.dot(xj_ref[...].astype(jnp.bfloat16), r_ref[...],
                 preferred_element_type=jnp.float32)
  msg_ref[...] = jnp.dot((xj_b * we).astype(jnp.bfloat16), s_ref[...],
                         preferred_element_type=jnp.float32)


def _gru_body(aggp_ref, degp_ref, h_ref, bconv_ref, wih_ref, whh_ref,
              bih_ref, bhh_ref, out_ref):
  agg = aggp_ref[0] + aggp_ref[1]
  deg = degp_ref[0][:, 0:1] + degp_ref[1][:, 0:1]
  deg = jnp.maximum(deg, 1.0)
  m = jax.nn.relu(agg / deg + bconv_ref[...])
  h = h_ref[...]
  gi = jnp.dot(m, wih_ref[...], preferred_element_type=jnp.float32) + bih_ref[...]
  gh = jnp.dot(h, whh_ref[...], preferred_element_type=jnp.float32) + bhh_ref[...]
  r = jax.nn.sigmoid(gi[:, 0:D] + gh[:, 0:D])
  z = jax.nn.sigmoid(gi[:, D:2 * D] + gh[:, D:2 * D])
  n = jnp.tanh(gi[:, 2 * D:3 * D] + r * gh[:, 2 * D:3 * D])
  out_ref[...] = (1.0 - z) * n + z * h


def _set2set_body(h_ref, batch_ref, wih_ref, whh_ref, bih_ref, bhh_ref,
                  q_ref):
  out = h_ref[...]
  onehot = (batch_ref[...] == lax.broadcasted_iota(
      jnp.int32, (N_PAD, B), 1)).astype(jnp.float32)
  valid = jnp.sum(onehot, axis=1, keepdims=True)
  hl = jnp.zeros((B, D), jnp.float32)
  cl = jnp.zeros((B, D), jnp.float32)
  q_star = jnp.zeros((B, 2 * D), jnp.float32)
  for _ in range(3):
    gates = (jnp.dot(q_star, wih_ref[...], preferred_element_type=jnp.float32)
             + bih_ref[...]
             + jnp.dot(hl, whh_ref[...], preferred_element_type=jnp.float32)
             + bhh_ref[...])
    gi_ = gates[:, 0:D]
    gf_ = gates[:, D:2 * D]
    gg_ = gates[:, 2 * D:3 * D]
    go_ = gates[:, 3 * D:4 * D]
    cl = jax.nn.sigmoid(gf_) * cl + jax.nn.sigmoid(gi_) * jnp.tanh(gg_)
    hl = jax.nn.sigmoid(go_) * jnp.tanh(cl)
    qb = jnp.dot(onehot, hl, preferred_element_type=jnp.float32)
    e = jnp.sum(out * qb, axis=1, keepdims=True)
    x_masked = jnp.where(onehot > 0.0, e, -1e30)
    emax = jnp.max(x_masked, axis=0, keepdims=True)
    emax_b = jnp.sum(onehot * emax, axis=1, keepdims=True)
    a = jnp.exp(e - emax_b) * valid
    s = jnp.sum(onehot * a, axis=0, keepdims=True)
    s_b = jnp.sum(onehot * s, axis=1, keepdims=True)
    a = a / (s_b + 1e-16)
    rvec = lax.dot_general(onehot, a * out, (((0,), (0,)), ((), ())),
                           preferred_element_type=jnp.float32)
    q_star = jnp.concatenate([hl, rvec], axis=1)
  q_ref[...] = q_star


def _tc_lin0(x_p, w0t, b0):
  return pl.pallas_call(
      _lin0_body,
      grid=(NGB,),
      in_specs=[
          pl.BlockSpec((NB, F_IN), lambda i: (i, 0)),
          pl.BlockSpec((F_IN, D), lambda i: (0, 0)),
          pl.BlockSpec((1, D), lambda i: (0, 0)),
      ],
      out_specs=pl.BlockSpec((NB, D), lambda i: (i, 0)),
      out_shape=jax.ShapeDtypeStruct((N_PAD, D), jnp.float32),
  )(x_p, w0t, b0)


def _tc_msg(ea_p, xj, w1t, b1, w2t, b2, r_mat, s_mat):
  return pl.pallas_call(
      _msg_body,
      grid=(EGB,),
      in_specs=[
          pl.BlockSpec((NB, 8), lambda i: (i, 0)),
          pl.BlockSpec((NB, D), lambda i: (i, 0)),
          pl.BlockSpec((8, F_IN), lambda i: (0, 0)),
          pl.BlockSpec((1, F_IN), lambda i: (0, 0)),
          pl.BlockSpec((F_IN, D * D), lambda i: (0, 0)),
          pl.BlockSpec((1, D * D), lambda i: (0, 0)),
          pl.BlockSpec((D, D * D), lambda i: (0, 0)),
          pl.BlockSpec((D * D, D), lambda i: (0, 0)),
      ],
      out_specs=pl.BlockSpec((NB, D), lambda i: (i, 0)),
      out_shape=jax.ShapeDtypeStruct((E_PAD, D), jnp.float32),
  )(ea_p, xj, w1t, b1, w2t, b2, r_mat, s_mat)


def _tc_gru(aggp, degp, h, bconv, wih_t, whh_t, bih, bhh):
  return pl.pallas_call(
      _gru_body,
      grid=(NGB,),
      in_specs=[
          pl.BlockSpec((2, NB, D), lambda i: (0, i, 0)),
          pl.BlockSpec((2, NB, 16), lambda i: (0, i, 0)),
          pl.BlockSpec((NB, D), lambda i: (i, 0)),
          pl.BlockSpec((1, D), lambda i: (0, 0)),
          pl.BlockSpec((D, 3 * D), lambda i: (0, 0)),
          pl.BlockSpec((D, 3 * D), lambda i: (0, 0)),
          pl.BlockSpec((1, 3 * D), lambda i: (0, 0)),
          pl.BlockSpec((1, 3 * D), lambda i: (0, 0)),
      ],
      out_specs=pl.BlockSpec((NB, D), lambda i: (i, 0)),
      out_shape=jax.ShapeDtypeStruct((N_PAD, D), jnp.float32),
  )(aggp, degp, h, bconv, wih_t, whh_t, bih, bhh)


def _tc_set2set(h, batch2d, wih_t, whh_t, bih, bhh):
  return pl.pallas_call(
      _set2set_body,
      out_shape=jax.ShapeDtypeStruct((B, 2 * D), jnp.float32),
  )(h, batch2d, wih_t, whh_t, bih, bhh)


def kernel(x, edge_index, edge_attr, batch, W0, b0, W1, b1, W2, b2, b_conv,
           gru_Wih, gru_Whh, gru_bih, gru_bhh, ls_Wih, ls_Whh, ls_bih,
           ls_bhh):
  x_p = jnp.pad(x, ((0, N_PAD - N), (0, 0)))
  batch2d = jnp.pad(batch, (0, N_PAD - N), constant_values=B).reshape(N_PAD, 1)
  ea_p = jnp.pad(edge_attr, ((0, E_PAD - E), (0, 3)))
  src_t = jnp.pad(edge_index[0], (0, E_PAD - E)).reshape(NTILES, NCHUNK, CHUNK)
  dst_t = jnp.pad(edge_index[1], (0, E_PAD - E),
                  constant_values=N).reshape(NTILES, NCHUNK, CHUNK)

  w0t = W0.T
  b0r = b0.reshape(1, D)
  w1t = jnp.pad(W1, ((0, 0), (0, 3))).T
  b1r = b1.reshape(1, F_IN)
  w2t = W2.T.astype(jnp.bfloat16)
  b2r = b2.reshape(1, D * D)
  bconv = b_conv.reshape(1, D)
  gwih_t = gru_Wih.T
  gwhh_t = gru_Whh.T
  gbih = gru_bih.reshape(1, 3 * D)
  gbhh = gru_bhh.reshape(1, 3 * D)
  lwih_t = ls_Wih.T
  lwhh_t = ls_Whh.T
  lbih = ls_bih.reshape(1, 4 * D)
  lbhh = ls_bhh.reshape(1, 4 * D)

  eye = jnp.eye(D, dtype=jnp.bfloat16)
  r_mat = jnp.repeat(eye, D, axis=1)
  s_mat = jnp.tile(eye, (D, 1))

  zagg = jnp.zeros((ROWS_PER_SUB, D), jnp.float32)
  zdeg = jnp.zeros((ROWS_PER_SUB, 16), jnp.float32)
  ones = jnp.ones((CHUNK, 16), jnp.float32)

  h = _tc_lin0(x_p, w0t, b0r)

  degp = None
  for r in range(3):
    xj = _sc_gather(h, src_t)
    msg = _tc_msg(ea_p, xj, w1t, b1r, w2t, b2r, r_mat, s_mat)
    if r == 0:
      aggp, degp = _sc_scatter_deg(msg, dst_t, zagg, zdeg, ones)
    else:
      aggp, _ = _sc_scatter(msg, dst_t, zagg, zdeg, ones)
    h = _tc_gru(aggp, degp, h, bconv, gwih_t, gwhh_t, gbih, gbhh)

  q_star = _tc_set2set(h, batch2d, lwih_t, lwhh_t, lbih, lbhh)
  return q_star, h[:N]


# speedup vs baseline: 1.0754x; 1.0002x over previous
import functools

import jax
import jax.numpy as jnp
from jax import lax
from jax.experimental import pallas as pl
from jax.experimental.pallas import tpu as pltpu
from jax.experimental.pallas import tpu_sc as plsc

N = 10000
E = 160000
F_IN = 128
D = 32
B = 64

N_PAD = 10240
E_PAD = 163840
NTILES = 32
EPT = E_PAD // NTILES
CHUNK = 128
NCHUNK = EPT // CHUNK
NB = 1280
NGB = N_PAD // NB
EGB = E_PAD // NB
ROWS_PER_SUB = N_PAD // 16

def _sc_mesh():
  return plsc.VectorSubcoreMesh(
      core_axis_name="c", subcore_axis_name="s", num_cores=2, num_subcores=16)


NBUF = 4


def _gather_body(h_hbm, src_hbm, xj_hbm, idx_v, row_v, gsem, wsem):
  cid = lax.axis_index("c")
  sid = lax.axis_index("s")
  wid = cid * 16 + sid
  pltpu.sync_copy(src_hbm.at[wid], idx_v)

  def superbatch(k, carry):
    for g in range(2):
      base = (k * 2 + g) * NBUF

      @pl.when(k > 0)
      def _drain():
        for _ in range(NBUF):
          pltpu.make_async_copy(row_v.at[g, 0], xj_hbm.at[pl.ds(0, CHUNK)],
                                wsem).wait()

      descs = [
          pltpu.async_copy(h_hbm.at[idx_v.at[base + b]], row_v.at[g, b], gsem)
          for b in range(NBUF)
      ]
      for dsc in descs:
        dsc.wait()
      for b in range(NBUF):
        pltpu.async_copy(
            row_v.at[g, b],
            xj_hbm.at[pl.ds((wid * NCHUNK + base + b) * CHUNK, CHUNK)], wsem)
    return carry

  lax.fori_loop(0, NCHUNK // (2 * NBUF), superbatch, 0)
  for _ in range(2 * NBUF):
    pltpu.make_async_copy(row_v.at[0, 0], xj_hbm.at[pl.ds(0, CHUNK)],
                          wsem).wait()


@functools.lru_cache(maxsize=None)
def _sc_gather_kernel():
  return pl.kernel(
      _gather_body,
      out_type=jax.ShapeDtypeStruct((E_PAD, D), jnp.float32),
      mesh=_sc_mesh(),
      compiler_params=pltpu.CompilerParams(use_tc_tiling_on_sc=False),
      scratch_types=[
          pltpu.VMEM((NCHUNK, CHUNK), jnp.int32),
          pltpu.VMEM((2, NBUF, CHUNK, D), jnp.float32),
          pltpu.SemaphoreType.DMA,
          pltpu.SemaphoreType.DMA,
      ],
  )


def _sc_gather(h, src_t):
  return _sc_gather_kernel()(h, src_t)


def _scatter_body(msg_hbm, dst_hbm, zagg_hbm, zdeg_hbm, ones_hbm,
                  aggp_hbm, degp_hbm, idx_v, msg_v, ones_v, agg_sh, deg_sh,
                  rsem, with_deg):
  cid = lax.axis_index("c")
  sid = lax.axis_index("s")
  wid = cid * 16 + sid
  pltpu.sync_copy(zagg_hbm, agg_sh.at[pl.ds(sid * ROWS_PER_SUB, ROWS_PER_SUB)])
  if with_deg:
    pltpu.sync_copy(zdeg_hbm,
                    deg_sh.at[pl.ds(sid * ROWS_PER_SUB, ROWS_PER_SUB)])
    pltpu.sync_copy(ones_hbm, ones_v)
  pltpu.sync_copy(dst_hbm.at[wid], idx_v)
  plsc.subcore_barrier()

  for p in range(4):
    pltpu.async_copy(msg_hbm.at[pl.ds((wid * NCHUNK + p) * CHUNK, CHUNK)],
                     msg_v.at[p], rsem)

  def quad(k, carry):
    for p in range(4):
      j = 4 * k + p
      pltpu.make_async_copy(msg_hbm.at[pl.ds(0, CHUNK)], msg_v.at[p],
                            rsem).wait()
      pltpu.sync_copy(msg_v.at[p], agg_sh.at[idx_v.at[j]], add=True)
      if with_deg:
        pltpu.sync_copy(ones_v, deg_sh.at[idx_v.at[j]], add=True)

      @pl.when(j + 4 < NCHUNK)
      def _prefetch():
        pltpu.async_copy(
            msg_hbm.at[pl.ds((wid * NCHUNK + j + 4) * CHUNK, CHUNK)],
            msg_v.at[p], rsem)
    return carry

  lax.fori_loop(0, NCHUNK // 4, quad, 0)
  plsc.subcore_barrier()
  sl = pl.ds(sid * ROWS_PER_SUB, ROWS_PER_SUB)
  pltpu.sync_copy(agg_sh.at[sl], aggp_hbm.at[cid].at[sl])
  if with_deg:
    pltpu.sync_copy(deg_sh.at[sl], degp_hbm.at[cid].at[sl])


@functools.lru_cache(maxsize=None)
def _make_scatter(with_deg):
  return pl.kernel(
      functools.partial(_scatter_body, with_deg=with_deg),
      out_type=(
          jax.ShapeDtypeStruct((2, N_PAD, D), jnp.float32),
          jax.ShapeDtypeStruct((2, N_PAD, 16), jnp.float32),
      ),
      mesh=_sc_mesh(),
      compiler_params=pltpu.CompilerParams(use_tc_tiling_on_sc=False),
      scratch_types=[
          pltpu.VMEM((NCHUNK, CHUNK), jnp.int32),
          pltpu.VMEM((4, CHUNK, D), jnp.float32),
          pltpu.VMEM((CHUNK, 16), jnp.float32),
          pltpu.VMEM_SHARED((N_PAD, D), jnp.float32),
          pltpu.VMEM_SHARED((N_PAD, 16), jnp.float32),
          pltpu.SemaphoreType.DMA,
      ],
  )


def _sc_scatter_deg(msg, dst_t, zagg, zdeg, ones):
  return _make_scatter(True)(msg, dst_t, zagg, zdeg, ones)


def _sc_scatter(msg, dst_t, zagg, zdeg, ones):
  return _make_scatter(False)(msg, dst_t, zagg, zdeg, ones)


def _lin0_body(x_ref, w0t_ref, b0_ref, out_ref):
  out_ref[...] = jax.nn.relu(
      jnp.dot(x_ref[...], w0t_ref[...], preferred_element_type=jnp.float32)
      + b0_ref[...])


def _msg_body(ea_ref, xj_ref, w1t_ref, b1_ref, w2t_ref, b2_ref, r_ref, s_ref,
              msg_ref):
  he = jax.nn.relu(
      jnp.dot(ea_ref[...], w1t_ref[...], preferred_element_type=jnp.float32)
      + b1_ref[...])
  we = (jnp.dot(he.astype(jnp.bfloat16), w2t_ref[...],
                preferred_element_type=jnp.float32)
        + b2_ref[...]).astype(jnp.bfloat16)
  xj_b = jnp.dot(xj_ref[...].astype(jnp.bfloat16), r_ref[...],
                 preferred_element_type=jnp.float32).astype(jnp.bfloat16)
  msg_ref[...] = jnp.dot(xj_b * we, s_ref[...],
                         preferred_element_type=jnp.float32)


def _gru_body(aggp_ref, degp_ref, h_ref, bconv_ref, wih_ref, whh_ref,
              bih_ref, bhh_ref, out_ref):
  agg = aggp_ref[0] + aggp_ref[1]
  deg = degp_ref[0][:, 0:1] + degp_ref[1][:, 0:1]
  deg = jnp.maximum(deg, 1.0)
  m = jax.nn.relu(agg / deg + bconv_ref[...])
  h = h_ref[...]
  gi = jnp.dot(m, wih_ref[...], preferred_element_type=jnp.float32) + bih_ref[...]
  gh = jnp.dot(h, whh_ref[...], preferred_element_type=jnp.float32) + bhh_ref[...]
  r = jax.nn.sigmoid(gi[:, 0:D] + gh[:, 0:D])
  z = jax.nn.sigmoid(gi[:, D:2 * D] + gh[:, D:2 * D])
  n = jnp.tanh(gi[:, 2 * D:3 * D] + r * gh[:, 2 * D:3 * D])
  out_ref[...] = (1.0 - z) * n + z * h


def _set2set_body(h_ref, batch_ref, wih_ref, whh_ref, bih_ref, bhh_ref,
                  q_ref):
  out = h_ref[...]
  onehot = (batch_ref[...] == lax.broadcasted_iota(
      jnp.int32, (N_PAD, B), 1)).astype(jnp.float32)
  valid = jnp.sum(onehot, axis=1, keepdims=True)
  hl = jnp.zeros((B, D), jnp.float32)
  cl = jnp.zeros((B, D), jnp.float32)
  q_star = jnp.zeros((B, 2 * D), jnp.float32)
  for _ in range(3):
    gates = (jnp.dot(q_star, wih_ref[...], preferred_element_type=jnp.float32)
             + bih_ref[...]
             + jnp.dot(hl, whh_ref[...], preferred_element_type=jnp.float32)
             + bhh_ref[...])
    gi_ = gates[:, 0:D]
    gf_ = gates[:, D:2 * D]
    gg_ = gates[:, 2 * D:3 * D]
    go_ = gates[:, 3 * D:4 * D]
    cl = jax.nn.sigmoid(gf_) * cl + jax.nn.sigmoid(gi_) * jnp.tanh(gg_)
    hl = jax.nn.sigmoid(go_) * jnp.tanh(cl)
    qb = jnp.dot(onehot, hl, preferred_element_type=jnp.float32)
    e = jnp.sum(out * qb, axis=1, keepdims=True)
    x_masked = jnp.where(onehot > 0.0, e, -1e30)
    emax = jnp.max(x_masked, axis=0, keepdims=True)
    emax_b = jnp.sum(onehot * emax, axis=1, keepdims=True)
    a = jnp.exp(e - emax_b) * valid
    s = jnp.sum(onehot * a, axis=0, keepdims=True)
    s_b = jnp.sum(onehot * s, axis=1, keepdims=True)
    a = a / (s_b + 1e-16)
    rvec = lax.dot_general(onehot, a * out, (((0,), (0,)), ((), ())),
                           preferred_element_type=jnp.float32)
    q_star = jnp.concatenate([hl, rvec], axis=1)
  q_ref[...] = q_star


def _tc_lin0(x_p, w0t, b0):
  return pl.pallas_call(
      _lin0_body,
      grid=(NGB,),
      in_specs=[
          pl.BlockSpec((NB, F_IN), lambda i: (i, 0)),
          pl.BlockSpec((F_IN, D), lambda i: (0, 0)),
          pl.BlockSpec((1, D), lambda i: (0, 0)),
      ],
      out_specs=pl.BlockSpec((NB, D), lambda i: (i, 0)),
      out_shape=jax.ShapeDtypeStruct((N_PAD, D), jnp.float32),
  )(x_p, w0t, b0)


def _tc_msg(ea_p, xj, w1t, b1, w2t, b2, r_mat, s_mat):
  return pl.pallas_call(
      _msg_body,
      grid=(EGB,),
      in_specs=[
          pl.BlockSpec((NB, 8), lambda i: (i, 0)),
          pl.BlockSpec((NB, D), lambda i: (i, 0)),
          pl.BlockSpec((8, F_IN), lambda i: (0, 0)),
          pl.BlockSpec((1, F_IN), lambda i: (0, 0)),
          pl.BlockSpec((F_IN, D * D), lambda i: (0, 0)),
          pl.BlockSpec((1, D * D), lambda i: (0, 0)),
          pl.BlockSpec((D, D * D), lambda i: (0, 0)),
          pl.BlockSpec((D * D, D), lambda i: (0, 0)),
      ],
      out_specs=pl.BlockSpec((NB, D), lambda i: (i, 0)),
      out_shape=jax.ShapeDtypeStruct((E_PAD, D), jnp.float32),
  )(ea_p, xj, w1t, b1, w2t, b2, r_mat, s_mat)


def _tc_gru(aggp, degp, h, bconv, wih_t, whh_t, bih, bhh):
  return pl.pallas_call(
      _gru_body,
      grid=(NGB,),
      in_specs=[
          pl.BlockSpec((2, NB, D), lambda i: (0, i, 0)),
          pl.BlockSpec((2, NB, 16), lambda i: (0, i, 0)),
          pl.BlockSpec((NB, D), lambda i: (i, 0)),
          pl.BlockSpec((1, D), lambda i: (0, 0)),
          pl.BlockSpec((D, 3 * D), lambda i: (0, 0)),
          pl.BlockSpec((D, 3 * D), lambda i: (0, 0)),
          pl.BlockSpec((1, 3 * D), lambda i: (0, 0)),
          pl.BlockSpec((1, 3 * D), lambda i: (0, 0)),
      ],
      out_specs=pl.BlockSpec((NB, D), lambda i: (i, 0)),
      out_shape=jax.ShapeDtypeStruct((N_PAD, D), jnp.float32),
  )(aggp, degp, h, bconv, wih_t, whh_t, bih, bhh)


def _tc_set2set(h, batch2d, wih_t, whh_t, bih, bhh):
  return pl.pallas_call(
      _set2set_body,
      out_shape=jax.ShapeDtypeStruct((B, 2 * D), jnp.float32),
  )(h, batch2d, wih_t, whh_t, bih, bhh)


def kernel(x, edge_index, edge_attr, batch, W0, b0, W1, b1, W2, b2, b_conv,
           gru_Wih, gru_Whh, gru_bih, gru_bhh, ls_Wih, ls_Whh, ls_bih,
           ls_bhh):
  x_p = jnp.pad(x, ((0, N_PAD - N), (0, 0)))
  batch2d = jnp.pad(batch, (0, N_PAD - N), constant_values=B).reshape(N_PAD, 1)
  ea_p = jnp.pad(edge_attr, ((0, E_PAD - E), (0, 3)))
  src_t = jnp.pad(edge_index[0], (0, E_PAD - E)).reshape(NTILES, NCHUNK, CHUNK)
  dst_t = jnp.pad(edge_index[1], (0, E_PAD - E),
                  constant_values=N).reshape(NTILES, NCHUNK, CHUNK)

  w0t = W0.T
  b0r = b0.reshape(1, D)
  w1t = jnp.pad(W1, ((0, 0), (0, 3))).T
  b1r = b1.reshape(1, F_IN)
  w2t = W2.T.astype(jnp.bfloat16)
  b2r = b2.reshape(1, D * D)
  bconv = b_conv.reshape(1, D)
  gwih_t = gru_Wih.T
  gwhh_t = gru_Whh.T
  gbih = gru_bih.reshape(1, 3 * D)
  gbhh = gru_bhh.reshape(1, 3 * D)
  lwih_t = ls_Wih.T
  lwhh_t = ls_Whh.T
  lbih = ls_bih.reshape(1, 4 * D)
  lbhh = ls_bhh.reshape(1, 4 * D)

  eye = jnp.eye(D, dtype=jnp.bfloat16)
  r_mat = jnp.repeat(eye, D, axis=1)
  s_mat = jnp.tile(eye, (D, 1))

  zagg = jnp.zeros((ROWS_PER_SUB, D), jnp.float32)
  zdeg = jnp.zeros((ROWS_PER_SUB, 16), jnp.float32)
  ones = jnp.ones((CHUNK, 16), jnp.float32)

  h = _tc_lin0(x_p, w0t, b0r)

  degp = None
  for r in range(3):
    xj = _sc_gather(h, src_t)
    msg = _tc_msg(ea_p, xj, w1t, b1r, w2t, b2r, r_mat, s_mat)
    if r == 0:
      aggp, degp = _sc_scatter_deg(msg, dst_t, zagg, zdeg, ones)
    else:
      aggp, _ = _sc_scatter(msg, dst_t, zagg, zdeg, ones)
    h = _tc_gru(aggp, degp, h, bconv, gwih_t, gwhh_t, gbih, gbhh)

  q_star = _tc_set2set(h, batch2d, lwih_t, lwhh_t, lbih, lbhh)
  return q_star, h[:N]


# speedup vs baseline: 1.1155x; 1.0373x over previous
import functools

import jax
import jax.numpy as jnp
from jax import lax
from jax.experimental import pallas as pl
from jax.experimental.pallas import tpu as pltpu
from jax.experimental.pallas import tpu_sc as plsc

N = 10000
E = 160000
F_IN = 128
D = 32
B = 64

N_PAD = 10240
E_PAD = 163840
NTILES = 32
EPT = E_PAD // NTILES
CHUNK = 128
NCHUNK = EPT // CHUNK
NB = 2560
NGB = N_PAD // NB
EGB = E_PAD // NB
ROWS_PER_SUB = N_PAD // 16

def _sc_mesh():
  return plsc.VectorSubcoreMesh(
      core_axis_name="c", subcore_axis_name="s", num_cores=2, num_subcores=16)


NBUF = 4


def _gather_body(h_hbm, src_hbm, xj_hbm, idx_v, row_v, gsem, wsem):
  cid = lax.axis_index("c")
  sid = lax.axis_index("s")
  wid = cid * 16 + sid
  pltpu.sync_copy(src_hbm.at[wid], idx_v)

  def superbatch(k, carry):
    for g in range(2):
      base = (k * 2 + g) * NBUF

      @pl.when(k > 0)
      def _drain():
        for _ in range(NBUF):
          pltpu.make_async_copy(row_v.at[g, 0], xj_hbm.at[pl.ds(0, CHUNK)],
                                wsem).wait()

      descs = [
          pltpu.async_copy(h_hbm.at[idx_v.at[base + b]], row_v.at[g, b], gsem)
          for b in range(NBUF)
      ]
      for dsc in descs:
        dsc.wait()
      for b in range(NBUF):
        pltpu.async_copy(
            row_v.at[g, b],
            xj_hbm.at[pl.ds((wid * NCHUNK + base + b) * CHUNK, CHUNK)], wsem)
    return carry

  lax.fori_loop(0, NCHUNK // (2 * NBUF), superbatch, 0)
  for _ in range(2 * NBUF):
    pltpu.make_async_copy(row_v.at[0, 0], xj_hbm.at[pl.ds(0, CHUNK)],
                          wsem).wait()


@functools.lru_cache(maxsize=None)
def _sc_gather_kernel():
  return pl.kernel(
      _gather_body,
      out_type=jax.ShapeDtypeStruct((E_PAD, D), jnp.float32),
      mesh=_sc_mesh(),
      compiler_params=pltpu.CompilerParams(use_tc_tiling_on_sc=False),
      scratch_types=[
          pltpu.VMEM((NCHUNK, CHUNK), jnp.int32),
          pltpu.VMEM((2, NBUF, CHUNK, D), jnp.float32),
          pltpu.SemaphoreType.DMA,
          pltpu.SemaphoreType.DMA,
      ],
  )


def _sc_gather(h, src_t):
  return _sc_gather_kernel()(h, src_t)


def _scatter_body(msg_hbm, dst_hbm, zagg_hbm, zdeg_hbm, ones_hbm,
                  aggp_hbm, degp_hbm, idx_v, msg_v, ones_v, agg_sh, deg_sh,
                  rsem, with_deg):
  cid = lax.axis_index("c")
  sid = lax.axis_index("s")
  wid = cid * 16 + sid
  pltpu.sync_copy(zagg_hbm, agg_sh.at[pl.ds(sid * ROWS_PER_SUB, ROWS_PER_SUB)])
  if with_deg:
    pltpu.sync_copy(zdeg_hbm,
                    deg_sh.at[pl.ds(sid * ROWS_PER_SUB, ROWS_PER_SUB)])
    pltpu.sync_copy(ones_hbm, ones_v)
  pltpu.sync_copy(dst_hbm.at[wid], idx_v)
  plsc.subcore_barrier()

  for p in range(4):
    pltpu.async_copy(msg_hbm.at[pl.ds((wid * NCHUNK + p) * CHUNK, CHUNK)],
                     msg_v.at[p], rsem)

  def quad(k, carry):
    for p in range(4):
      j = 4 * k + p
      pltpu.make_async_copy(msg_hbm.at[pl.ds(0, CHUNK)], msg_v.at[p],
                            rsem).wait()
      pltpu.sync_copy(msg_v.at[p], agg_sh.at[idx_v.at[j]], add=True)
      if with_deg:
        pltpu.sync_copy(ones_v, deg_sh.at[idx_v.at[j]], add=True)

      @pl.when(j + 4 < NCHUNK)
      def _prefetch():
        pltpu.async_copy(
            msg_hbm.at[pl.ds((wid * NCHUNK + j + 4) * CHUNK, CHUNK)],
            msg_v.at[p], rsem)
    return carry

  lax.fori_loop(0, NCHUNK // 4, quad, 0)
  plsc.subcore_barrier()
  sl = pl.ds(sid * ROWS_PER_SUB, ROWS_PER_SUB)
  pltpu.sync_copy(agg_sh.at[sl], aggp_hbm.at[cid].at[sl])
  if with_deg:
    pltpu.sync_copy(deg_sh.at[sl], degp_hbm.at[cid].at[sl])


@functools.lru_cache(maxsize=None)
def _make_scatter(with_deg):
  return pl.kernel(
      functools.partial(_scatter_body, with_deg=with_deg),
      out_type=(
          jax.ShapeDtypeStruct((2, N_PAD, D), jnp.float32),
          jax.ShapeDtypeStruct((2, N_PAD, 16), jnp.float32),
      ),
      mesh=_sc_mesh(),
      compiler_params=pltpu.CompilerParams(use_tc_tiling_on_sc=False),
      scratch_types=[
          pltpu.VMEM((NCHUNK, CHUNK), jnp.int32),
          pltpu.VMEM((4, CHUNK, D), jnp.float32),
          pltpu.VMEM((CHUNK, 16), jnp.float32),
          pltpu.VMEM_SHARED((N_PAD, D), jnp.float32),
          pltpu.VMEM_SHARED((N_PAD, 16), jnp.float32),
          pltpu.SemaphoreType.DMA,
      ],
  )


def _sc_scatter_deg(msg, dst_t, zagg, zdeg, ones):
  return _make_scatter(True)(msg, dst_t, zagg, zdeg, ones)


def _sc_scatter(msg, dst_t, zagg, zdeg, ones):
  return _make_scatter(False)(msg, dst_t, zagg, zdeg, ones)


def _lin0_body(x_ref, w0t_ref, b0_ref, out_ref):
  out_ref[...] = jax.nn.relu(
      jnp.dot(x_ref[...], w0t_ref[...], preferred_element_type=jnp.float32)
      + b0_ref[...])


def _msg_body(ea_ref, xj_ref, w1t_ref, b1_ref, w2t_ref, b2_ref, r_ref, s_ref,
              msg_ref):
  he = jax.nn.relu(
      jnp.dot(ea_ref[...], w1t_ref[...], preferred_element_type=jnp.float32)
      + b1_ref[...])
  we = (jnp.dot(he.astype(jnp.bfloat16), w2t_ref[...],
                preferred_element_type=jnp.float32)
        + b2_ref[...]).astype(jnp.bfloat16)
  xj_b = jnp.dot(xj_ref[...].astype(jnp.bfloat16), r_ref[...],
                 preferred_element_type=jnp.float32).astype(jnp.bfloat16)
  msg_ref[...] = jnp.dot(xj_b * we, s_ref[...],
                         preferred_element_type=jnp.float32)


def _gru_body(aggp_ref, degp_ref, h_ref, bconv_ref, wih_ref, whh_ref,
              bih_ref, bhh_ref, out_ref):
  agg = aggp_ref[0] + aggp_ref[1]
  deg = degp_ref[0][:, 0:1] + degp_ref[1][:, 0:1]
  deg = jnp.maximum(deg, 1.0)
  m = jax.nn.relu(agg / deg + bconv_ref[...])
  h = h_ref[...]
  gi = jnp.dot(m, wih_ref[...], preferred_element_type=jnp.float32) + bih_ref[...]
  gh = jnp.dot(h, whh_ref[...], preferred_element_type=jnp.float32) + bhh_ref[...]
  r = jax.nn.sigmoid(gi[:, 0:D] + gh[:, 0:D])
  z = jax.nn.sigmoid(gi[:, D:2 * D] + gh[:, D:2 * D])
  n = jnp.tanh(gi[:, 2 * D:3 * D] + r * gh[:, 2 * D:3 * D])
  out_ref[...] = (1.0 - z) * n + z * h


def _set2set_body(h_ref, batch_ref, wih_ref, whh_ref, bih_ref, bhh_ref,
                  q_ref):
  out = h_ref[...]
  onehot = (batch_ref[...] == lax.broadcasted_iota(
      jnp.int32, (N_PAD, B), 1)).astype(jnp.float32)
  valid = jnp.sum(onehot, axis=1, keepdims=True)
  hl = jnp.zeros((B, D), jnp.float32)
  cl = jnp.zeros((B, D), jnp.float32)
  q_star = jnp.zeros((B, 2 * D), jnp.float32)
  for _ in range(3):
    gates = (jnp.dot(q_star, wih_ref[...], preferred_element_type=jnp.float32)
             + bih_ref[...]
             + jnp.dot(hl, whh_ref[...], preferred_element_type=jnp.float32)
             + bhh_ref[...])
    gi_ = gates[:, 0:D]
    gf_ = gates[:, D:2 * D]
    gg_ = gates[:, 2 * D:3 * D]
    go_ = gates[:, 3 * D:4 * D]
    cl = jax.nn.sigmoid(gf_) * cl + jax.nn.sigmoid(gi_) * jnp.tanh(gg_)
    hl = jax.nn.sigmoid(go_) * jnp.tanh(cl)
    qb = jnp.dot(onehot, hl, preferred_element_type=jnp.float32)
    e = jnp.sum(out * qb, axis=1, keepdims=True)
    x_masked = jnp.where(onehot > 0.0, e, -1e30)
    emax = jnp.max(x_masked, axis=0, keepdims=True)
    emax_b = jnp.sum(onehot * emax, axis=1, keepdims=True)
    a = jnp.exp(e - emax_b) * valid
    s = jnp.sum(onehot * a, axis=0, keepdims=True)
    s_b = jnp.sum(onehot * s, axis=1, keepdims=True)
    a = a / (s_b + 1e-16)
    rvec = lax.dot_general(onehot, a * out, (((0,), (0,)), ((), ())),
                           preferred_element_type=jnp.float32)
    q_star = jnp.concatenate([hl, rvec], axis=1)
  q_ref[...] = q_star


def _tc_lin0(x_p, w0t, b0):
  return pl.pallas_call(
      _lin0_body,
      grid=(NGB,),
      in_specs=[
          pl.BlockSpec((NB, F_IN), lambda i: (i, 0)),
          pl.BlockSpec((F_IN, D), lambda i: (0, 0)),
          pl.BlockSpec((1, D), lambda i: (0, 0)),
      ],
      out_specs=pl.BlockSpec((NB, D), lambda i: (i, 0)),
      out_shape=jax.ShapeDtypeStruct((N_PAD, D), jnp.float32),
  )(x_p, w0t, b0)


def _tc_msg(ea_p, xj, w1t, b1, w2t, b2, r_mat, s_mat):
  return pl.pallas_call(
      _msg_body,
      grid=(EGB,),
      in_specs=[
          pl.BlockSpec((NB, 8), lambda i: (i, 0)),
          pl.BlockSpec((NB, D), lambda i: (i, 0)),
          pl.BlockSpec((8, F_IN), lambda i: (0, 0)),
          pl.BlockSpec((1, F_IN), lambda i: (0, 0)),
          pl.BlockSpec((F_IN, D * D), lambda i: (0, 0)),
          pl.BlockSpec((1, D * D), lambda i: (0, 0)),
          pl.BlockSpec((D, D * D), lambda i: (0, 0)),
          pl.BlockSpec((D * D, D), lambda i: (0, 0)),
      ],
      out_specs=pl.BlockSpec((NB, D), lambda i: (i, 0)),
      out_shape=jax.ShapeDtypeStruct((E_PAD, D), jnp.float32),
  )(ea_p, xj, w1t, b1, w2t, b2, r_mat, s_mat)


def _tc_gru(aggp, degp, h, bconv, wih_t, whh_t, bih, bhh):
  return pl.pallas_call(
      _gru_body,
      grid=(NGB,),
      in_specs=[
          pl.BlockSpec((2, NB, D), lambda i: (0, i, 0)),
          pl.BlockSpec((2, NB, 16), lambda i: (0, i, 0)),
          pl.BlockSpec((NB, D), lambda i: (i, 0)),
          pl.BlockSpec((1, D), lambda i: (0, 0)),
          pl.BlockSpec((D, 3 * D), lambda i: (0, 0)),
          pl.BlockSpec((D, 3 * D), lambda i: (0, 0)),
          pl.BlockSpec((1, 3 * D), lambda i: (0, 0)),
          pl.BlockSpec((1, 3 * D), lambda i: (0, 0)),
      ],
      out_specs=pl.BlockSpec((NB, D), lambda i: (i, 0)),
      out_shape=jax.ShapeDtypeStruct((N_PAD, D), jnp.float32),
  )(aggp, degp, h, bconv, wih_t, whh_t, bih, bhh)


def _tc_set2set(h, batch2d, wih_t, whh_t, bih, bhh):
  return pl.pallas_call(
      _set2set_body,
      out_shape=jax.ShapeDtypeStruct((B, 2 * D), jnp.float32),
  )(h, batch2d, wih_t, whh_t, bih, bhh)


def kernel(x, edge_index, edge_attr, batch, W0, b0, W1, b1, W2, b2, b_conv,
           gru_Wih, gru_Whh, gru_bih, gru_bhh, ls_Wih, ls_Whh, ls_bih,
           ls_bhh):
  x_p = jnp.pad(x, ((0, N_PAD - N), (0, 0)))
  batch2d = jnp.pad(batch, (0, N_PAD - N), constant_values=B).reshape(N_PAD, 1)
  ea_p = jnp.pad(edge_attr, ((0, E_PAD - E), (0, 3)))
  src_t = jnp.pad(edge_index[0], (0, E_PAD - E)).reshape(NTILES, NCHUNK, CHUNK)
  dst_t = jnp.pad(edge_index[1], (0, E_PAD - E),
                  constant_values=N).reshape(NTILES, NCHUNK, CHUNK)

  w0t = W0.T
  b0r = b0.reshape(1, D)
  w1t = jnp.pad(W1, ((0, 0), (0, 3))).T
  b1r = b1.reshape(1, F_IN)
  w2t = W2.T.astype(jnp.bfloat16)
  b2r = b2.reshape(1, D * D)
  bconv = b_conv.reshape(1, D)
  gwih_t = gru_Wih.T
  gwhh_t = gru_Whh.T
  gbih = gru_bih.reshape(1, 3 * D)
  gbhh = gru_bhh.reshape(1, 3 * D)
  lwih_t = ls_Wih.T
  lwhh_t = ls_Whh.T
  lbih = ls_bih.reshape(1, 4 * D)
  lbhh = ls_bhh.reshape(1, 4 * D)

  eye = jnp.eye(D, dtype=jnp.bfloat16)
  r_mat = jnp.repeat(eye, D, axis=1)
  s_mat = jnp.tile(eye, (D, 1))

  zagg = jnp.zeros((ROWS_PER_SUB, D), jnp.float32)
  zdeg = jnp.zeros((ROWS_PER_SUB, 16), jnp.float32)
  ones = jnp.ones((CHUNK, 16), jnp.float32)

  h = _tc_lin0(x_p, w0t, b0r)

  degp = None
  for r in range(3):
    xj = _sc_gather(h, src_t)
    msg = _tc_msg(ea_p, xj, w1t, b1r, w2t, b2r, r_mat, s_mat)
    if r == 0:
      aggp, degp = _sc_scatter_deg(msg, dst_t, zagg, zdeg, ones)
    else:
      aggp, _ = _sc_scatter(msg, dst_t, zagg, zdeg, ones)
    h = _tc_gru(aggp, degp, h, bconv, gwih_t, gwhh_t, gbih, gbhh)

  q_star = _tc_set2set(h, batch2d, lwih_t, lwhh_t, lbih, lbhh)
  return q_star, h[:N]
